# Initial kernel scaffold; baseline (speedup 1.0000x reference)
import jax
import jax.numpy as jnp
from jax.experimental import pallas as pl


def kernel(features, edge_index, W, b):
    raise NotImplementedError("write your pallas kernel here")


# speedup vs baseline: 5.4513x; 5.4513x over previous
import functools

import jax
import jax.numpy as jnp
from jax import lax
from jax.experimental import pallas as pl
from jax.experimental.pallas import tpu as pltpu
from jax.experimental.pallas import tpu_sc as plsc

N = 10000
E = 320000
F = 64
NP = 10240
NC = 2
NS = 16
NW = NC * NS
EW = E // NW
CH = 80
NCHUNK = EW // CH
RT = NP // NS

_mesh = plsc.VectorSubcoreMesh(core_axis_name="c", subcore_axis_name="s")
_sc_params = pltpu.CompilerParams(use_tc_tiling_on_sc=False)


@functools.partial(
    pl.kernel,
    out_type=jax.ShapeDtypeStruct((NC * NP,), jnp.float32),
    mesh=_mesh,
    compiler_params=_sc_params,
    scratch_types=[
        pltpu.VMEM_SHARED((NP,), jnp.float32),
        pltpu.VMEM((RT,), jnp.float32),
        pltpu.VMEM((CH,), jnp.int32),
        pltpu.VMEM((CH,), jnp.float32),
    ],
)
def _deg_sc(dstp_hbm, out_hbm, acc, stage, didx, ones):
    cid = lax.axis_index("c")
    sid = lax.axis_index("s")
    wid = sid * NC + cid
    z16 = jnp.zeros((16,), jnp.float32)
    o16 = jnp.ones((16,), jnp.float32)

    def zl(i, c):
        stage[pl.ds(i * 16, 16)] = z16
        return c

    lax.fori_loop(0, RT // 16, zl, 0)

    def ol(i, c):
        ones[pl.ds(i * 16, 16)] = o16
        return c

    lax.fori_loop(0, CH // 16, ol, 0)

    row0 = sid * RT
    pltpu.sync_copy(stage, acc.at[pl.ds(row0, RT)])
    plsc.subcore_barrier()

    base = wid * EW

    def chunk(i, c):
        pltpu.sync_copy(dstp_hbm.at[pl.ds(base + i * CH, CH)], didx)
        pltpu.sync_copy(ones, acc.at[didx], add=True)
        return c

    lax.fori_loop(0, NCHUNK, chunk, 0)
    plsc.subcore_barrier()

    pltpu.sync_copy(acc.at[pl.ds(row0, RT)], stage)
    pltpu.sync_copy(stage, out_hbm.at[pl.ds(cid * NP + row0, RT)])


@functools.partial(
    pl.kernel,
    out_type=jax.ShapeDtypeStruct((NC * NP, F), jnp.float32),
    mesh=_mesh,
    compiler_params=_sc_params,
    scratch_types=[
        pltpu.VMEM_SHARED((NP, F), jnp.float32),
        pltpu.VMEM((64, F), jnp.float32),
        pltpu.VMEM((CH,), jnp.int32),
        pltpu.VMEM((CH,), jnp.int32),
        pltpu.VMEM((CH, F), jnp.float32),
        pltpu.SemaphoreType.DMA,
    ],
)
def _prop_sc(t_hbm, src_hbm, dstp_hbm, out_hbm, acc, stage, sidx, didx, rows,
             sem):
    cid = lax.axis_index("c")
    sid = lax.axis_index("s")
    wid = sid * NC + cid
    z16 = jnp.zeros((16,), jnp.float32)

    def zl(i, c):
        stage[i // (F // 16), pl.ds((i % (F // 16)) * 16, 16)] = z16
        return c

    lax.fori_loop(0, 64 * (F // 16), zl, 0)

    row0 = sid * RT

    def zacc(j, c):
        pltpu.sync_copy(stage, acc.at[pl.ds(row0 + j * 64, 64)])
        return c

    lax.fori_loop(0, RT // 64, zacc, 0)
    plsc.subcore_barrier()

    base = wid * EW

    def chunk(i, c):
        off = base + i * CH
        pltpu.sync_copy(src_hbm.at[pl.ds(off, CH)], sidx)
        pltpu.async_copy(t_hbm.at[sidx], rows, sem).wait()
        pltpu.sync_copy(dstp_hbm.at[pl.ds(off, CH)], didx)
        pltpu.sync_copy(rows, acc.at[didx], add=True)
        return c

    lax.fori_loop(0, NCHUNK, chunk, 0)
    plsc.subcore_barrier()

    out0 = cid * NP + row0

    def wb(j, c):
        pltpu.sync_copy(acc.at[pl.ds(row0 + j * 64, 64)], stage)
        pltpu.sync_copy(stage, out_hbm.at[pl.ds(out0 + j * 64, 64)])
        return c

    lax.fori_loop(0, RT // 64, wb, 0)


def _prep_body(src_ref, dst_ref, out_ref):
    s = src_ref[...]
    d = dst_ref[...]
    out_ref[...] = jnp.where(s != d, d, jnp.int32(N))


def _mm_body(x_ref, w_ref, y_ref):
    y_ref[...] = lax.dot_general(
        x_ref[...], w_ref[...], (((1,), (1,)), ((), ())),
        preferred_element_type=jnp.float32)


def _comb1_body(d0_ref, d1_ref, y_ref, t1_ref, nrm_ref):
    deg = d0_ref[...] + d1_ref[...] + 1.0
    nrm = lax.rsqrt(jnp.maximum(deg, 1.0))
    nrm_ref[...] = nrm
    t1_ref[...] = y_ref[...] * nrm


def _comb2_body(acc_ref, t1_ref, nrm_ref, t2_ref):
    nrm = nrm_ref[...]
    t2_ref[...] = (acc_ref[0] + acc_ref[1] + t1_ref[...]) * (nrm * nrm)


def _final_body(acc_ref, t2_ref, nrm_ref, b_ref, o_ref):
    o_ref[...] = (acc_ref[0] + acc_ref[1] + t2_ref[...]) * nrm_ref[...] \
        + b_ref[...]


def _tc_call(body, out_shape, *args):
    return pl.pallas_call(body, out_shape=out_shape)(*args)


def kernel(features, edge_index, W, b):
    src = edge_index[0]
    dst = edge_index[1]

    dstp = _tc_call(
        _prep_body, jax.ShapeDtypeStruct((2500, 128), jnp.int32),
        src.reshape(2500, 128), dst.reshape(2500, 128)).reshape(E)

    deg_parts = _deg_sc(dstp)
    d0 = deg_parts[:NP].reshape(NP, 1)
    d1 = deg_parts[NP:].reshape(NP, 1)

    xp = jnp.pad(features, ((0, NP - N), (0, 0)))
    yp = _tc_call(_mm_body, jax.ShapeDtypeStruct((NP, F), jnp.float32),
                  xp, W)

    t1, nrm = _tc_call(
        _comb1_body,
        (jax.ShapeDtypeStruct((NP, F), jnp.float32),
         jax.ShapeDtypeStruct((NP, 1), jnp.float32)),
        d0, d1, yp)

    acc1 = _prop_sc(t1, src, dstp).reshape(NC, NP, F)
    t2 = _tc_call(_comb2_body, jax.ShapeDtypeStruct((NP, F), jnp.float32),
                  acc1, t1, nrm)

    acc2 = _prop_sc(t2, src, dstp).reshape(NC, NP, F)
    out = _tc_call(_final_body, jax.ShapeDtypeStruct((NP, F), jnp.float32),
                   acc2, t2, nrm, b.reshape(1, F))

    return out[:N]


# speedup vs baseline: 6.6368x; 1.2175x over previous
import functools

import jax
import jax.numpy as jnp
from jax import lax
from jax.experimental import pallas as pl
from jax.experimental.pallas import tpu as pltpu
from jax.experimental.pallas import tpu_sc as plsc

N = 10000
E = 320000
F = 64
NP = 10240
NC = 2
NS = 16
NW = NC * NS
EW = E // NW
CHP = 128
NCHP = 80
EP = NW * NCHP * CHP
RT = NP // NS

_mesh = plsc.VectorSubcoreMesh(core_axis_name="c", subcore_axis_name="s")
_sc_params = pltpu.CompilerParams(use_tc_tiling_on_sc=False)


@functools.partial(
    pl.kernel,
    out_type=jax.ShapeDtypeStruct((NC * NP,), jnp.float32),
    mesh=_mesh,
    compiler_params=_sc_params,
    scratch_types=[
        pltpu.VMEM_SHARED((NP,), jnp.float32),
        pltpu.VMEM((RT,), jnp.float32),
        pltpu.VMEM((NCHP, CHP), jnp.int32),
        pltpu.VMEM((CHP,), jnp.float32),
        pltpu.SemaphoreType.DMA,
    ],
)
def _deg_sc(dstp_hbm, out_hbm, acc, stage, didx, ones, isem):
    cid = lax.axis_index("c")
    sid = lax.axis_index("s")
    wid = sid * NC + cid

    c0 = wid * NCHP
    pltpu.async_copy(dstp_hbm.at[pl.ds(c0, NCHP)], didx, isem)

    z16 = jnp.zeros((16,), jnp.float32)
    o16 = jnp.ones((16,), jnp.float32)

    def zl(i, c):
        stage[pl.ds(i * 16, 16)] = z16
        return c

    lax.fori_loop(0, RT // 16, zl, 0)

    def ol(i, c):
        ones[pl.ds(i * 16, 16)] = o16
        return c

    lax.fori_loop(0, CHP // 16, ol, 0)

    row0 = sid * RT
    pltpu.sync_copy(stage, acc.at[pl.ds(row0, RT)])
    pltpu.make_async_copy(dstp_hbm.at[pl.ds(c0, NCHP)], didx, isem).wait()
    plsc.subcore_barrier()

    def chunk(i, c):
        pltpu.sync_copy(ones, acc.at[didx.at[i]], add=True)
        return c

    lax.fori_loop(0, NCHP, chunk, 0)
    plsc.subcore_barrier()

    pltpu.sync_copy(acc.at[pl.ds(row0, RT)], stage)
    pltpu.sync_copy(stage, out_hbm.at[pl.ds(cid * NP + row0, RT)])


@functools.partial(
    pl.kernel,
    out_type=jax.ShapeDtypeStruct((NC * NP, F), jnp.float32),
    mesh=_mesh,
    compiler_params=_sc_params,
    scratch_types=[
        pltpu.VMEM_SHARED((NP, F), jnp.float32),
        pltpu.VMEM((64, F), jnp.float32),
        pltpu.VMEM((NCHP, CHP), jnp.int32),
        pltpu.VMEM((NCHP, CHP), jnp.int32),
        pltpu.VMEM((CHP, F), jnp.float32),
        pltpu.VMEM((CHP, F), jnp.float32),
        pltpu.SemaphoreType.DMA,
        pltpu.SemaphoreType.DMA,
        pltpu.SemaphoreType.DMA,
    ],
)
def _prop_sc(t_hbm, src_hbm, dstp_hbm, out_hbm, acc, stage, sidx, didx,
             rows0, rows1, sem0, sem1, isem):
    cid = lax.axis_index("c")
    sid = lax.axis_index("s")
    wid = sid * NC + cid

    c0 = wid * NCHP
    pltpu.async_copy(src_hbm.at[pl.ds(c0, NCHP)], sidx, isem)
    pltpu.async_copy(dstp_hbm.at[pl.ds(c0, NCHP)], didx, isem)

    z16 = jnp.zeros((16,), jnp.float32)

    def zl(i, c):
        stage[i // (F // 16), pl.ds((i % (F // 16)) * 16, 16)] = z16
        return c

    lax.fori_loop(0, 64 * (F // 16), zl, 0)

    row0 = sid * RT

    def zacc(j, c):
        pltpu.sync_copy(stage, acc.at[pl.ds(row0 + j * 64, 64)])
        return c

    lax.fori_loop(0, RT // 64, zacc, 0)

    pltpu.make_async_copy(src_hbm.at[pl.ds(c0, NCHP)], sidx, isem).wait()
    pltpu.make_async_copy(dstp_hbm.at[pl.ds(c0, NCHP)], didx, isem).wait()

    pltpu.async_copy(t_hbm.at[sidx.at[0]], rows0, sem0)
    plsc.subcore_barrier()

    def body(j, c):
        i0 = 2 * j
        pltpu.async_copy(t_hbm.at[sidx.at[i0 + 1]], rows1, sem1)
        pltpu.make_async_copy(t_hbm.at[sidx.at[0]], rows0, sem0).wait()
        pltpu.sync_copy(rows0, acc.at[didx.at[i0]], add=True)

        @pl.when(j < NCHP // 2 - 1)
        def _():
            pltpu.async_copy(t_hbm.at[sidx.at[i0 + 2]], rows0, sem0)

        pltpu.make_async_copy(t_hbm.at[sidx.at[1]], rows1, sem1).wait()
        pltpu.sync_copy(rows1, acc.at[didx.at[i0 + 1]], add=True)
        return c

    lax.fori_loop(0, NCHP // 2, body, 0)
    plsc.subcore_barrier()

    out0 = cid * NP + row0

    def wb(j, c):
        pltpu.sync_copy(acc.at[pl.ds(row0 + j * 64, 64)], stage)
        pltpu.sync_copy(stage, out_hbm.at[pl.ds(out0 + j * 64, 64)])
        return c

    lax.fori_loop(0, RT // 64, wb, 0)


def _prep_body(src_ref, dst_ref, out_ref):
    s = src_ref[...]
    d = dst_ref[...]
    out_ref[...] = jnp.where(s != d, d, jnp.int32(N))


def _mm_body(x_ref, w_ref, y_ref):
    y_ref[...] = lax.dot_general(
        x_ref[...], w_ref[...], (((1,), (1,)), ((), ())),
        preferred_element_type=jnp.float32)


def _comb1_body(d0_ref, d1_ref, y_ref, t1_ref, nrm_ref):
    deg = d0_ref[...] + d1_ref[...] + 1.0
    nrm = lax.rsqrt(jnp.maximum(deg, 1.0))
    nrm_ref[...] = nrm
    t1_ref[...] = y_ref[...] * nrm


def _comb2_body(acc_ref, t1_ref, nrm_ref, t2_ref):
    nrm = nrm_ref[...]
    t2_ref[...] = (acc_ref[0] + acc_ref[1] + t1_ref[...]) * (nrm * nrm)


def _final_body(acc_ref, t2_ref, nrm_ref, b_ref, o_ref):
    o_ref[...] = (acc_ref[0] + acc_ref[1] + t2_ref[...]) * nrm_ref[...] \
        + b_ref[...]


def _tc_call(body, out_shape, *args):
    return pl.pallas_call(body, out_shape=out_shape)(*args)


def kernel(features, edge_index, W, b):
    src = edge_index[0]
    dst = edge_index[1]

    dstp = _tc_call(
        _prep_body, jax.ShapeDtypeStruct((2500, 128), jnp.int32),
        src.reshape(2500, 128), dst.reshape(2500, 128)).reshape(E)

    pad = EP - E
    dstp2 = jnp.concatenate(
        [dstp, jnp.full((pad,), N, jnp.int32)]).reshape(EP // CHP, CHP)
    src2 = jnp.concatenate(
        [src, jnp.zeros((pad,), jnp.int32)]).reshape(EP // CHP, CHP)

    deg_parts = _deg_sc(dstp2)
    d0 = deg_parts[:NP].reshape(NP, 1)
    d1 = deg_parts[NP:].reshape(NP, 1)

    xp = jnp.pad(features, ((0, NP - N), (0, 0)))
    yp = _tc_call(_mm_body, jax.ShapeDtypeStruct((NP, F), jnp.float32),
                  xp, W)

    t1, nrm = _tc_call(
        _comb1_body,
        (jax.ShapeDtypeStruct((NP, F), jnp.float32),
         jax.ShapeDtypeStruct((NP, 1), jnp.float32)),
        d0, d1, yp)

    acc1 = _prop_sc(t1, src2, dstp2).reshape(NC, NP, F)
    t2 = _tc_call(_comb2_body, jax.ShapeDtypeStruct((NP, F), jnp.float32),
                  acc1, t1, nrm)

    acc2 = _prop_sc(t2, src2, dstp2).reshape(NC, NP, F)
    out = _tc_call(_final_body, jax.ShapeDtypeStruct((NP, F), jnp.float32),
                   acc2, t2, nrm, b.reshape(1, F))

    return out[:N]


# speedup vs baseline: 6.7689x; 1.0199x over previous
import functools

import jax
import jax.numpy as jnp
from jax import lax
from jax.experimental import pallas as pl
from jax.experimental.pallas import tpu as pltpu
from jax.experimental.pallas import tpu_sc as plsc

N = 10000
E = 320000
F = 64
NP = 10240
NC = 2
NS = 16
NW = NC * NS
EW = E // NW
CHP = 128
NCHP = 80
EP = NW * NCHP * CHP
RT = NP // NS

_mesh = plsc.VectorSubcoreMesh(core_axis_name="c", subcore_axis_name="s")
_sc_params = pltpu.CompilerParams(use_tc_tiling_on_sc=False)


@functools.partial(
    pl.kernel,
    out_type=jax.ShapeDtypeStruct((NC * NP,), jnp.float32),
    mesh=_mesh,
    compiler_params=_sc_params,
    scratch_types=[
        pltpu.VMEM_SHARED((NP,), jnp.float32),
        pltpu.VMEM((RT,), jnp.float32),
        pltpu.VMEM((NCHP, CHP), jnp.int32),
        pltpu.VMEM((CHP,), jnp.float32),
        pltpu.SemaphoreType.DMA,
    ],
)
def _deg_sc(dstp_hbm, out_hbm, acc, stage, didx, ones, isem):
    cid = lax.axis_index("c")
    sid = lax.axis_index("s")
    wid = sid * NC + cid

    c0 = wid * NCHP
    pltpu.async_copy(dstp_hbm.at[pl.ds(c0, NCHP)], didx, isem)

    z16 = jnp.zeros((16,), jnp.float32)
    o16 = jnp.ones((16,), jnp.float32)

    def zl(i, c):
        stage[pl.ds(i * 16, 16)] = z16
        return c

    lax.fori_loop(0, RT // 16, zl, 0)

    def ol(i, c):
        ones[pl.ds(i * 16, 16)] = o16
        return c

    lax.fori_loop(0, CHP // 16, ol, 0)

    row0 = sid * RT
    pltpu.sync_copy(stage, acc.at[pl.ds(row0, RT)])
    pltpu.make_async_copy(dstp_hbm.at[pl.ds(c0, NCHP)], didx, isem).wait()
    plsc.subcore_barrier()

    def chunk(i, c):
        pltpu.sync_copy(ones, acc.at[didx.at[i]], add=True)
        return c

    lax.fori_loop(0, NCHP, chunk, 0)
    plsc.subcore_barrier()

    pltpu.sync_copy(acc.at[pl.ds(row0, RT)], stage)
    pltpu.sync_copy(stage, out_hbm.at[pl.ds(cid * NP + row0, RT)])


@functools.partial(
    pl.kernel,
    out_type=jax.ShapeDtypeStruct((NC * NP, F), jnp.float32),
    mesh=_mesh,
    compiler_params=_sc_params,
    scratch_types=[
        pltpu.VMEM_SHARED((NP, F), jnp.float32),
        pltpu.VMEM((64, F), jnp.float32),
        pltpu.VMEM((NCHP, CHP), jnp.int32),
        pltpu.VMEM((NCHP, CHP), jnp.int32),
        pltpu.VMEM((CHP, F), jnp.float32),
        pltpu.VMEM((CHP, F), jnp.float32),
        pltpu.SemaphoreType.DMA,
        pltpu.SemaphoreType.DMA,
        pltpu.SemaphoreType.DMA,
    ],
)
def _prop_sc(t_hbm, src_hbm, dstp_hbm, out_hbm, acc, stage, sidx, didx,
             rows0, rows1, sem0, sem1, isem):
    cid = lax.axis_index("c")
    sid = lax.axis_index("s")
    wid = sid * NC + cid

    c0 = wid * NCHP
    pltpu.async_copy(src_hbm.at[pl.ds(c0, NCHP)], sidx, isem)
    pltpu.async_copy(dstp_hbm.at[pl.ds(c0, NCHP)], didx, isem)

    z16 = jnp.zeros((16,), jnp.float32)

    def zl(i, c):
        stage[i // (F // 16), pl.ds((i % (F // 16)) * 16, 16)] = z16
        return c

    lax.fori_loop(0, 64 * (F // 16), zl, 0)

    row0 = sid * RT

    def zacc(j, c):
        pltpu.sync_copy(stage, acc.at[pl.ds(row0 + j * 64, 64)])
        return c

    lax.fori_loop(0, RT // 64, zacc, 0)

    pltpu.make_async_copy(src_hbm.at[pl.ds(c0, NCHP)], sidx, isem).wait()
    pltpu.make_async_copy(dstp_hbm.at[pl.ds(c0, NCHP)], didx, isem).wait()

    pltpu.async_copy(t_hbm.at[sidx.at[0]], rows0, sem0)
    plsc.subcore_barrier()

    def body(j, c):
        i0 = 2 * j
        pltpu.async_copy(t_hbm.at[sidx.at[i0 + 1]], rows1, sem1)
        pltpu.make_async_copy(t_hbm.at[sidx.at[0]], rows0, sem0).wait()
        pltpu.sync_copy(rows0, acc.at[didx.at[i0]], add=True)

        @pl.when(j < NCHP // 2 - 1)
        def _():
            pltpu.async_copy(t_hbm.at[sidx.at[i0 + 2]], rows0, sem0)

        pltpu.make_async_copy(t_hbm.at[sidx.at[1]], rows1, sem1).wait()
        pltpu.sync_copy(rows1, acc.at[didx.at[i0 + 1]], add=True)
        return c

    lax.fori_loop(0, NCHP // 2, body, 0)
    plsc.subcore_barrier()

    out0 = cid * NP + row0

    def wb(j, c):
        pltpu.sync_copy(acc.at[pl.ds(row0 + j * 64, 64)], stage)
        pltpu.sync_copy(stage, out_hbm.at[pl.ds(out0 + j * 64, 64)])
        return c

    lax.fori_loop(0, RT // 64, wb, 0)


def _prep_body(src_ref, dst_ref, out_ref):
    s = src_ref[...]
    d = dst_ref[...]
    out_ref[...] = jnp.where(s != d, d, jnp.int32(N))


def _mm_body(x_ref, w_ref, y_ref):
    y_ref[...] = lax.dot_general(
        x_ref[...], w_ref[...], (((1,), (1,)), ((), ())),
        preferred_element_type=jnp.float32)


def _comb1_body(d0_ref, d1_ref, y_ref, t1_ref, nrm_ref):
    deg = d0_ref[...] + d1_ref[...] + 1.0
    nrm = lax.rsqrt(jnp.maximum(deg, 1.0))
    nrm_ref[...] = nrm
    t1_ref[...] = y_ref[...] * nrm


def _comb2_body(acc_ref, t1_ref, nrm_ref, t2_ref):
    nrm = nrm_ref[...]
    t2_ref[...] = (acc_ref[0] + acc_ref[1] + t1_ref[...]) * (nrm * nrm)


def _final_body(acc_ref, t2_ref, nrm_ref, b_ref, o_ref):
    o_ref[...] = (acc_ref[0] + acc_ref[1] + t2_ref[...]) * nrm_ref[...] \
        + b_ref[...]


def _tc_call(body, out_shape, *args):
    return pl.pallas_call(body, out_shape=out_shape)(*args)


def kernel(features, edge_index, W, b):
    src = edge_index[0]
    dst = edge_index[1]

    dstp = _tc_call(
        _prep_body, jax.ShapeDtypeStruct((2500, 128), jnp.int32),
        src.reshape(2500, 128), dst.reshape(2500, 128)).reshape(E)

    ppw = NCHP * CHP - EW
    pad_dst = jnp.broadcast_to(N + jnp.arange(ppw, dtype=jnp.int32),
                               (NW, ppw))
    pad_src = jnp.zeros((NW, ppw), jnp.int32)
    dstp2 = jnp.concatenate(
        [dstp.reshape(NW, EW), pad_dst], axis=1).reshape(EP // CHP, CHP)
    src2 = jnp.concatenate(
        [src.reshape(NW, EW), pad_src], axis=1).reshape(EP // CHP, CHP)

    deg_parts = _deg_sc(dstp2)
    d0 = deg_parts[:NP].reshape(NP, 1)
    d1 = deg_parts[NP:].reshape(NP, 1)

    xp = jnp.pad(features, ((0, NP - N), (0, 0)))
    yp = _tc_call(_mm_body, jax.ShapeDtypeStruct((NP, F), jnp.float32),
                  xp, W)

    t1, nrm = _tc_call(
        _comb1_body,
        (jax.ShapeDtypeStruct((NP, F), jnp.float32),
         jax.ShapeDtypeStruct((NP, 1), jnp.float32)),
        d0, d1, yp)

    acc1 = _prop_sc(t1, src2, dstp2).reshape(NC, NP, F)
    t2 = _tc_call(_comb2_body, jax.ShapeDtypeStruct((NP, F), jnp.float32),
                  acc1, t1, nrm)

    acc2 = _prop_sc(t2, src2, dstp2).reshape(NC, NP, F)
    out = _tc_call(_final_body, jax.ShapeDtypeStruct((NP, F), jnp.float32),
                   acc2, t2, nrm, b.reshape(1, F))

    return out[:N]


# speedup vs baseline: 6.9481x; 1.0265x over previous
import functools

import jax
import jax.numpy as jnp
from jax import lax
from jax.experimental import pallas as pl
from jax.experimental.pallas import tpu as pltpu
from jax.experimental.pallas import tpu_sc as plsc

N = 10000
E = 320000
F = 64
NP = 10240
NC = 2
NS = 16
NW = NC * NS
EW = E // NW
CHP = 128
NCHP = 80
EP = NW * NCHP * CHP
NBUF = 4
RT = NP // NS

_mesh = plsc.VectorSubcoreMesh(core_axis_name="c", subcore_axis_name="s")
_sc_params = pltpu.CompilerParams(use_tc_tiling_on_sc=False)


@functools.partial(
    pl.kernel,
    out_type=jax.ShapeDtypeStruct((NC * NP,), jnp.float32),
    mesh=_mesh,
    compiler_params=_sc_params,
    scratch_types=[
        pltpu.VMEM_SHARED((NP,), jnp.float32),
        pltpu.VMEM((RT,), jnp.float32),
        pltpu.VMEM((NCHP, CHP), jnp.int32),
        pltpu.VMEM((CHP,), jnp.float32),
        pltpu.SemaphoreType.DMA,
    ],
)
def _deg_sc(dstp_hbm, out_hbm, acc, stage, didx, ones, isem):
    cid = lax.axis_index("c")
    sid = lax.axis_index("s")
    wid = sid * NC + cid

    c0 = wid * NCHP
    pltpu.async_copy(dstp_hbm.at[pl.ds(c0, NCHP)], didx, isem)

    z16 = jnp.zeros((16,), jnp.float32)
    o16 = jnp.ones((16,), jnp.float32)

    def zl(i, c):
        stage[pl.ds(i * 16, 16)] = z16
        return c

    lax.fori_loop(0, RT // 16, zl, 0)

    def ol(i, c):
        ones[pl.ds(i * 16, 16)] = o16
        return c

    lax.fori_loop(0, CHP // 16, ol, 0)

    row0 = sid * RT
    pltpu.sync_copy(stage, acc.at[pl.ds(row0, RT)])
    pltpu.make_async_copy(dstp_hbm.at[pl.ds(c0, NCHP)], didx, isem).wait()
    plsc.subcore_barrier()

    def chunk(i, c):
        pltpu.sync_copy(ones, acc.at[didx.at[i]], add=True)
        return c

    lax.fori_loop(0, NCHP, chunk, 0)
    plsc.subcore_barrier()

    pltpu.sync_copy(acc.at[pl.ds(row0, RT)], stage)
    pltpu.sync_copy(stage, out_hbm.at[pl.ds(cid * NP + row0, RT)])


@functools.partial(
    pl.kernel,
    out_type=jax.ShapeDtypeStruct((NC * NP, F), jnp.float32),
    mesh=_mesh,
    compiler_params=_sc_params,
    scratch_types=[
        pltpu.VMEM_SHARED((NP, F), jnp.float32),
        pltpu.VMEM((64, F), jnp.float32),
        pltpu.VMEM((NCHP, CHP), jnp.int32),
        pltpu.VMEM((NCHP, CHP), jnp.int32),
        [pltpu.VMEM((CHP, F), jnp.float32) for _ in range(NBUF)],
        [pltpu.SemaphoreType.DMA for _ in range(NBUF)],
        [pltpu.SemaphoreType.DMA for _ in range(NBUF)],
        pltpu.SemaphoreType.DMA,
    ],
)
def _prop_sc(t_hbm, src_hbm, dstp_hbm, out_hbm, acc, stage, sidx, didx,
             rows, gsem, ssem, isem):
    cid = lax.axis_index("c")
    sid = lax.axis_index("s")
    wid = sid * NC + cid

    c0 = wid * NCHP
    pltpu.async_copy(src_hbm.at[pl.ds(c0, NCHP)], sidx, isem)
    pltpu.async_copy(dstp_hbm.at[pl.ds(c0, NCHP)], didx, isem)

    z16 = jnp.zeros((16,), jnp.float32)

    def zl(i, c):
        stage[i // (F // 16), pl.ds((i % (F // 16)) * 16, 16)] = z16
        return c

    lax.fori_loop(0, 64 * (F // 16), zl, 0)

    row0 = sid * RT

    def zacc(j, c):
        pltpu.sync_copy(stage, acc.at[pl.ds(row0 + j * 64, 64)])
        return c

    lax.fori_loop(0, RT // 64, zacc, 0)

    pltpu.make_async_copy(src_hbm.at[pl.ds(c0, NCHP)], sidx, isem).wait()
    pltpu.make_async_copy(dstp_hbm.at[pl.ds(c0, NCHP)], didx, isem).wait()

    for b in range(NBUF):
        pltpu.async_copy(t_hbm.at[sidx.at[b]], rows[b], gsem[b])
    plsc.subcore_barrier()

    def body(j, c):
        i0 = NBUF * j
        for b in range(NBUF):
            pltpu.make_async_copy(
                t_hbm.at[sidx.at[0]], rows[b], gsem[b]).wait()
            pltpu.async_copy(
                rows[b], acc.at[didx.at[i0 + b]], ssem[b], add=True)
        for b in range(NBUF):
            @pl.when(j < NCHP // NBUF - 1)
            def _(b=b):
                pltpu.make_async_copy(
                    rows[b], acc.at[didx.at[0]], ssem[b]).wait()
                pltpu.async_copy(
                    t_hbm.at[sidx.at[i0 + NBUF + b]], rows[b], gsem[b])
        return c

    lax.fori_loop(0, NCHP // NBUF, body, 0)
    for b in range(NBUF):
        pltpu.make_async_copy(rows[b], acc.at[didx.at[0]], ssem[b]).wait()
    plsc.subcore_barrier()

    out0 = cid * NP + row0

    def wb(j, c):
        pltpu.sync_copy(acc.at[pl.ds(row0 + j * 64, 64)], stage)
        pltpu.sync_copy(stage, out_hbm.at[pl.ds(out0 + j * 64, 64)])
        return c

    lax.fori_loop(0, RT // 64, wb, 0)


def _prep_body(src_ref, dst_ref, out_ref):
    s = src_ref[...]
    d = dst_ref[...]
    out_ref[...] = jnp.where(s != d, d, jnp.int32(N))


def _mm_body(x_ref, w_ref, y_ref):
    y_ref[...] = lax.dot_general(
        x_ref[...], w_ref[...], (((1,), (1,)), ((), ())),
        preferred_element_type=jnp.float32)


def _comb1_body(d0_ref, d1_ref, y_ref, t1_ref, nrm_ref):
    deg = d0_ref[...] + d1_ref[...] + 1.0
    nrm = lax.rsqrt(jnp.maximum(deg, 1.0))
    nrm_ref[...] = nrm
    t1_ref[...] = y_ref[...] * nrm


def _comb2_body(acc_ref, t1_ref, nrm_ref, t2_ref):
    nrm = nrm_ref[...]
    t2_ref[...] = (acc_ref[0] + acc_ref[1] + t1_ref[...]) * (nrm * nrm)


def _final_body(acc_ref, t2_ref, nrm_ref, b_ref, o_ref):
    o_ref[...] = (acc_ref[0] + acc_ref[1] + t2_ref[...]) * nrm_ref[...] \
        + b_ref[...]


def _tc_call(body, out_shape, *args):
    return pl.pallas_call(body, out_shape=out_shape)(*args)


def kernel(features, edge_index, W, b):
    src = edge_index[0]
    dst = edge_index[1]

    dstp = _tc_call(
        _prep_body, jax.ShapeDtypeStruct((2500, 128), jnp.int32),
        src.reshape(2500, 128), dst.reshape(2500, 128)).reshape(E)

    ppw = NCHP * CHP - EW
    pad_dst = jnp.broadcast_to(N + jnp.arange(ppw, dtype=jnp.int32),
                               (NW, ppw))
    pad_src = jnp.zeros((NW, ppw), jnp.int32)
    dstp2 = jnp.concatenate(
        [dstp.reshape(NW, EW), pad_dst], axis=1).reshape(EP // CHP, CHP)
    src2 = jnp.concatenate(
        [src.reshape(NW, EW), pad_src], axis=1).reshape(EP // CHP, CHP)

    deg_parts = _deg_sc(dstp2)
    d0 = deg_parts[:NP].reshape(NP, 1)
    d1 = deg_parts[NP:].reshape(NP, 1)

    xp = jnp.pad(features, ((0, NP - N), (0, 0)))
    yp = _tc_call(_mm_body, jax.ShapeDtypeStruct((NP, F), jnp.float32),
                  xp, W)

    t1, nrm = _tc_call(
        _comb1_body,
        (jax.ShapeDtypeStruct((NP, F), jnp.float32),
         jax.ShapeDtypeStruct((NP, 1), jnp.float32)),
        d0, d1, yp)

    acc1 = _prop_sc(t1, src2, dstp2).reshape(NC, NP, F)
    t2 = _tc_call(_comb2_body, jax.ShapeDtypeStruct((NP, F), jnp.float32),
                  acc1, t1, nrm)

    acc2 = _prop_sc(t2, src2, dstp2).reshape(NC, NP, F)
    out = _tc_call(_final_body, jax.ShapeDtypeStruct((NP, F), jnp.float32),
                   acc2, t2, nrm, b.reshape(1, F))

    return out[:N]


# speedup vs baseline: 12.5899x; 1.8120x over previous
import functools

import jax
import jax.numpy as jnp
from jax import lax
from jax.experimental import pallas as pl
from jax.experimental.pallas import tpu as pltpu
from jax.experimental.pallas import tpu_sc as plsc

N = 10000
E = 320000
F = 64
NP = 10240
NC = 2
NS = 16
NW = NC * NS
EW = E // NW
CHP = 128
NCHP = 80
EP = NW * NCHP * CHP
NBUF = 2
RT = NP // NS

_mesh = plsc.VectorSubcoreMesh(core_axis_name="c", subcore_axis_name="s")
_sc_params = pltpu.CompilerParams(use_tc_tiling_on_sc=False)


@functools.partial(
    pl.kernel,
    out_type=jax.ShapeDtypeStruct((NC * NP,), jnp.float32),
    mesh=_mesh,
    compiler_params=_sc_params,
    scratch_types=[
        pltpu.VMEM_SHARED((NP,), jnp.float32),
        pltpu.VMEM((RT,), jnp.float32),
        pltpu.VMEM((NCHP, CHP), jnp.int32),
        pltpu.VMEM((CHP,), jnp.float32),
        pltpu.SemaphoreType.DMA,
    ],
)
def _deg_sc(dstp_hbm, out_hbm, acc, stage, didx, ones, isem):
    cid = lax.axis_index("c")
    sid = lax.axis_index("s")
    wid = sid * NC + cid

    c0 = wid * NCHP
    pltpu.async_copy(dstp_hbm.at[pl.ds(c0, NCHP)], didx, isem)

    z16 = jnp.zeros((16,), jnp.float32)
    o16 = jnp.ones((16,), jnp.float32)

    def zl(i, c):
        stage[pl.ds(i * 16, 16)] = z16
        return c

    lax.fori_loop(0, RT // 16, zl, 0)

    def ol(i, c):
        ones[pl.ds(i * 16, 16)] = o16
        return c

    lax.fori_loop(0, CHP // 16, ol, 0)

    row0 = sid * RT
    pltpu.sync_copy(stage, acc.at[pl.ds(row0, RT)])
    pltpu.make_async_copy(dstp_hbm.at[pl.ds(c0, NCHP)], didx, isem).wait()
    plsc.subcore_barrier()

    def chunk(i, c):
        pltpu.sync_copy(ones, acc.at[didx.at[i]], add=True)
        return c

    lax.fori_loop(0, NCHP, chunk, 0)
    plsc.subcore_barrier()

    pltpu.sync_copy(acc.at[pl.ds(row0, RT)], stage)
    pltpu.sync_copy(stage, out_hbm.at[pl.ds(cid * NP + row0, RT)])


@functools.partial(
    pl.kernel,
    out_type=jax.ShapeDtypeStruct((NC * NP, F), jnp.float32),
    mesh=_mesh,
    compiler_params=_sc_params,
    scratch_types=[
        pltpu.VMEM_SHARED((NP, F), jnp.float32),
        pltpu.VMEM_SHARED((NP, F), jnp.float32),
        pltpu.VMEM((NCHP, CHP), jnp.int32),
        pltpu.VMEM((NCHP, CHP), jnp.int32),
        [pltpu.VMEM((CHP, F), jnp.float32) for _ in range(NBUF)],
        [pltpu.SemaphoreType.DMA for _ in range(NBUF)],
        [pltpu.SemaphoreType.DMA for _ in range(NBUF)],
        pltpu.SemaphoreType.DMA,
    ],
)
def _prop_sc(t_hbm, src_hbm, dstp_hbm, out_hbm, acc, tsh, sidx, didx,
             rows, gsem, ssem, isem):
    cid = lax.axis_index("c")
    sid = lax.axis_index("s")
    wid = sid * NC + cid

    c0 = wid * NCHP
    pltpu.async_copy(src_hbm.at[pl.ds(c0, NCHP)], sidx, isem)
    pltpu.async_copy(dstp_hbm.at[pl.ds(c0, NCHP)], didx, isem)

    z16 = jnp.zeros((16,), jnp.float32)

    def zl(i, c):
        rows[0][i // (F // 16), pl.ds((i % (F // 16)) * 16, 16)] = z16
        return c

    lax.fori_loop(0, CHP * (F // 16), zl, 0)

    row0 = sid * RT

    def zacc(j, c):
        pltpu.sync_copy(rows[0], acc.at[pl.ds(row0 + j * CHP, CHP)])
        return c

    lax.fori_loop(0, RT // CHP, zacc, 0)

    def st(j, c):
        r = row0 + j * CHP
        pltpu.sync_copy(t_hbm.at[pl.ds(r, CHP)], rows[1])
        pltpu.sync_copy(rows[1], tsh.at[pl.ds(r, CHP)])
        return c

    lax.fori_loop(0, RT // CHP, st, 0)

    pltpu.make_async_copy(src_hbm.at[pl.ds(c0, NCHP)], sidx, isem).wait()
    pltpu.make_async_copy(dstp_hbm.at[pl.ds(c0, NCHP)], didx, isem).wait()
    plsc.subcore_barrier()

    for b in range(NBUF):
        pltpu.async_copy(tsh.at[sidx.at[b]], rows[b], gsem[b])

    def body(j, c):
        i0 = NBUF * j
        for b in range(NBUF):
            pltpu.make_async_copy(
                tsh.at[sidx.at[0]], rows[b], gsem[b]).wait()
            pltpu.async_copy(
                rows[b], acc.at[didx.at[i0 + b]], ssem[b], add=True)
        for b in range(NBUF):
            @pl.when(j < NCHP // NBUF - 1)
            def _(b=b):
                pltpu.make_async_copy(
                    rows[b], acc.at[didx.at[0]], ssem[b]).wait()
                pltpu.async_copy(
                    tsh.at[sidx.at[i0 + NBUF + b]], rows[b], gsem[b])
        return c

    lax.fori_loop(0, NCHP // NBUF, body, 0)
    for b in range(NBUF):
        pltpu.make_async_copy(rows[b], acc.at[didx.at[0]], ssem[b]).wait()
    plsc.subcore_barrier()

    out0 = cid * NP + row0

    def wb(j, c):
        pltpu.sync_copy(acc.at[pl.ds(row0 + j * CHP, CHP)], rows[0])
        pltpu.sync_copy(rows[0], out_hbm.at[pl.ds(out0 + j * CHP, CHP)])
        return c

    lax.fori_loop(0, RT // CHP, wb, 0)


def _prep_body(src_ref, dst_ref, out_ref):
    s = src_ref[...]
    d = dst_ref[...]
    out_ref[...] = jnp.where(s != d, d, jnp.int32(N))


def _mm_body(x_ref, w_ref, y_ref):
    y_ref[...] = lax.dot_general(
        x_ref[...], w_ref[...], (((1,), (1,)), ((), ())),
        preferred_element_type=jnp.float32)


def _comb1_body(d0_ref, d1_ref, y_ref, t1_ref, nrm_ref):
    deg = d0_ref[...] + d1_ref[...] + 1.0
    nrm = lax.rsqrt(jnp.maximum(deg, 1.0))
    nrm_ref[...] = nrm
    t1_ref[...] = y_ref[...] * nrm


def _comb2_body(acc_ref, t1_ref, nrm_ref, t2_ref):
    nrm = nrm_ref[...]
    t2_ref[...] = (acc_ref[0] + acc_ref[1] + t1_ref[...]) * (nrm * nrm)


def _final_body(acc_ref, t2_ref, nrm_ref, b_ref, o_ref):
    o_ref[...] = (acc_ref[0] + acc_ref[1] + t2_ref[...]) * nrm_ref[...] \
        + b_ref[...]


def _tc_call(body, out_shape, *args):
    return pl.pallas_call(body, out_shape=out_shape)(*args)


def kernel(features, edge_index, W, b):
    src = edge_index[0]
    dst = edge_index[1]

    dstp = _tc_call(
        _prep_body, jax.ShapeDtypeStruct((2500, 128), jnp.int32),
        src.reshape(2500, 128), dst.reshape(2500, 128)).reshape(E)

    ppw = NCHP * CHP - EW
    pad_dst = jnp.broadcast_to(N + jnp.arange(ppw, dtype=jnp.int32),
                               (NW, ppw))
    pad_src = jnp.zeros((NW, ppw), jnp.int32)
    dstp2 = jnp.concatenate(
        [dstp.reshape(NW, EW), pad_dst], axis=1).reshape(EP // CHP, CHP)
    src2 = jnp.concatenate(
        [src.reshape(NW, EW), pad_src], axis=1).reshape(EP // CHP, CHP)

    deg_parts = _deg_sc(dstp2)
    d0 = deg_parts[:NP].reshape(NP, 1)
    d1 = deg_parts[NP:].reshape(NP, 1)

    xp = jnp.pad(features, ((0, NP - N), (0, 0)))
    yp = _tc_call(_mm_body, jax.ShapeDtypeStruct((NP, F), jnp.float32),
                  xp, W)

    t1, nrm = _tc_call(
        _comb1_body,
        (jax.ShapeDtypeStruct((NP, F), jnp.float32),
         jax.ShapeDtypeStruct((NP, 1), jnp.float32)),
        d0, d1, yp)

    acc1 = _prop_sc(t1, src2, dstp2).reshape(NC, NP, F)
    t2 = _tc_call(_comb2_body, jax.ShapeDtypeStruct((NP, F), jnp.float32),
                  acc1, t1, nrm)

    acc2 = _prop_sc(t2, src2, dstp2).reshape(NC, NP, F)
    out = _tc_call(_final_body, jax.ShapeDtypeStruct((NP, F), jnp.float32),
                   acc2, t2, nrm, b.reshape(1, F))

    return out[:N]
